# baseline (speedup 1.0000x reference)
import functools

import jax
import jax.numpy as jnp
import numpy as np
from jax.experimental import pallas as pl

TAU_ = 0.5
EPS_ = float(np.finfo(np.float32).tiny)
SUB_ = 8


def _concrete_row_kernel(n_cols, logits_ref, out_ref):
    row = pl.program_id(0)
    lane = n_cols // SUB_
    l = logits_ref[0]

    s_io = jax.lax.broadcasted_iota(jnp.int32, (SUB_, lane), 0)
    c_io = jax.lax.broadcasted_iota(jnp.int32, (SUB_, lane), 1)
    x1 = (s_io * lane + c_io + row * n_cols).astype(jnp.uint32)

    ks0 = jnp.uint32(0)
    ks1 = jnp.uint32(1)
    ks2 = jnp.uint32(0x1BD11BDA ^ 0 ^ 1)

    x0 = jnp.zeros_like(x1) + ks0
    x1 = x1 + ks1

    def rotl(v, d):
        return (v << jnp.uint32(d)) | (v >> jnp.uint32(32 - d))

    rots_a = (13, 15, 26, 6)
    rots_b = (17, 29, 16, 24)
    inject = ((ks1, ks2), (ks2, ks0), (ks0, ks1), (ks1, ks2), (ks2, ks0))
    for i in range(5):
        for r in (rots_a if i % 2 == 0 else rots_b):
            x0 = x0 + x1
            x1 = rotl(x1, r)
            x1 = x1 ^ x0
        ka, kb = inject[i]
        x0 = x0 + ka
        x1 = x1 + kb + jnp.uint32(i + 1)

    bits = x0 ^ x1

    fu = jax.lax.bitcast_convert_type(
        (bits >> jnp.uint32(9)) | jnp.uint32(0x3F800000), jnp.float32
    ) - jnp.float32(1.0)
    u = fu * jnp.float32(1.0 - EPS_) + jnp.float32(EPS_)
    u = jnp.maximum(jnp.float32(EPS_), u)

    t = jnp.log(u)
    w = (t * t) * jnp.exp(jnp.float32(1.0 / TAU_) * l)
    s = jnp.sum(w)
    out_ref[0] = w * (jnp.float32(1.0) / s)


def kernel(logits):
    rows, n_cols = logits.shape
    lane = n_cols // SUB_
    x3 = logits.reshape(rows, SUB_, lane)
    out = pl.pallas_call(
        functools.partial(_concrete_row_kernel, n_cols),
        grid=(rows,),
        in_specs=[pl.BlockSpec((1, SUB_, lane), lambda r: (r, 0, 0))],
        out_specs=pl.BlockSpec((1, SUB_, lane), lambda r: (r, 0, 0)),
        out_shape=jax.ShapeDtypeStruct((rows, SUB_, lane), jnp.float32),
    )(x3)
    return out.reshape(rows, n_cols)


# speedup vs baseline: 1.1236x; 1.1236x over previous
import functools

import jax
import jax.numpy as jnp
import numpy as np
from jax.experimental import pallas as pl

TAU_ = 0.5
EPS_ = float(np.finfo(np.float32).tiny)
SUB_ = 8
CHUNK_ = 512


def _weights(l, idx):
    ks0 = jnp.uint32(0)
    ks1 = jnp.uint32(1)
    ks2 = jnp.uint32(0x1BD11BDA ^ 0 ^ 1)

    x0 = jnp.zeros_like(idx)
    x1 = idx + ks1

    def rotl(v, d):
        return (v << jnp.uint32(d)) | (v >> jnp.uint32(32 - d))

    rots_a = (13, 15, 26, 6)
    rots_b = (17, 29, 16, 24)
    inject = ((ks1, ks2), (ks2, ks0), (ks0, ks1), (ks1, ks2), (ks2, ks0))
    for i in range(5):
        for r in (rots_a if i % 2 == 0 else rots_b):
            x0 = x0 + x1
            x1 = rotl(x1, r)
            x1 = x1 ^ x0
        ka, kb = inject[i]
        x0 = x0 + ka
        x1 = x1 + kb + jnp.uint32(i + 1)

    bits = x0 ^ x1

    fu = jax.lax.bitcast_convert_type(
        (bits >> jnp.uint32(9)) | jnp.uint32(0x3F800000), jnp.float32
    ) - jnp.float32(1.0)
    u = fu * jnp.float32(1.0 - EPS_) + jnp.float32(EPS_)
    u = jnp.maximum(jnp.float32(EPS_), u)

    t = jnp.log(u)
    return (t * t) * jnp.exp(jnp.float32(1.0 / TAU_) * l)


def _concrete_row_kernel(n_cols, logits_ref, out_ref):
    lane = n_cols // SUB_
    n_full = lane // CHUNK_
    tail = lane - n_full * CHUNK_
    row_base = pl.program_id(0) * n_cols

    s_io = jax.lax.broadcasted_iota(jnp.int32, (SUB_, CHUNK_), 0) * lane
    c_io = jax.lax.broadcasted_iota(jnp.int32, (SUB_, CHUNK_), 1)
    base_idx = s_io + c_io + row_base

    def pass1(i, acc):
        st = i * CHUNK_
        l = logits_ref[0, :, pl.ds(st, CHUNK_)]
        w = _weights(l, (base_idx + st).astype(jnp.uint32))
        out_ref[0, :, pl.ds(st, CHUNK_)] = w
        return acc + w

    acc = jax.lax.fori_loop(
        0, n_full, pass1, jnp.zeros((SUB_, CHUNK_), jnp.float32)
    )
    total = jnp.sum(acc)

    if tail:
        st = n_full * CHUNK_
        l = logits_ref[0, :, pl.ds(st, tail)]
        w = _weights(l, (base_idx[:, :tail] + st).astype(jnp.uint32))
        out_ref[0, :, pl.ds(st, tail)] = w
        total = total + jnp.sum(w)

    inv = jnp.float32(1.0) / total

    def pass2(i, carry):
        st = i * CHUNK_
        out_ref[0, :, pl.ds(st, CHUNK_)] = out_ref[0, :, pl.ds(st, CHUNK_)] * inv
        return carry

    jax.lax.fori_loop(0, n_full, pass2, 0)
    if tail:
        st = n_full * CHUNK_
        out_ref[0, :, pl.ds(st, tail)] = out_ref[0, :, pl.ds(st, tail)] * inv


def kernel(logits):
    rows, n_cols = logits.shape
    lane = n_cols // SUB_
    x3 = logits.reshape(rows, SUB_, lane)
    out = pl.pallas_call(
        functools.partial(_concrete_row_kernel, n_cols),
        grid=(rows,),
        in_specs=[pl.BlockSpec((1, SUB_, lane), lambda r: (r, 0, 0))],
        out_specs=pl.BlockSpec((1, SUB_, lane), lambda r: (r, 0, 0)),
        out_shape=jax.ShapeDtypeStruct((rows, SUB_, lane), jnp.float32),
    )(x3)
    return out.reshape(rows, n_cols)


# speedup vs baseline: 1.2367x; 1.1007x over previous
import functools

import jax
import jax.numpy as jnp
import numpy as np
from jax.experimental import pallas as pl

TAU_ = 0.5
EPS_ = float(np.finfo(np.float32).tiny)
SUB_ = 8
CHUNK_ = 1024
_EXP2_SCALE = float(2.0 / np.log(2.0))


def _weights(l, idx):
    ks0 = jnp.uint32(0)
    ks1 = jnp.uint32(1)
    ks2 = jnp.uint32(0x1BD11BDA ^ 0 ^ 1)

    x0 = jnp.zeros_like(idx)
    x1 = idx + ks1

    def rotl(v, d):
        return (v << jnp.uint32(d)) | (v >> jnp.uint32(32 - d))

    rots_a = (13, 15, 26, 6)
    rots_b = (17, 29, 16, 24)
    inject = ((ks1, ks2), (ks2, ks0), (ks0, ks1), (ks1, ks2), (ks2, ks0))
    for i in range(5):
        for r in (rots_a if i % 2 == 0 else rots_b):
            x0 = x0 + x1
            x1 = rotl(x1, r)
            x1 = x1 ^ x0
        ka, kb = inject[i]
        x0 = x0 + ka
        x1 = x1 + kb + jnp.uint32(i + 1)

    bits = x0 ^ x1

    fu = jax.lax.bitcast_convert_type(
        (bits >> jnp.uint32(9)) | jnp.uint32(0x3F800000), jnp.float32
    ) - jnp.float32(1.0)
    u = jnp.maximum(fu, jnp.float32(EPS_))

    t = jnp.log2(u)
    return (t * t) * jnp.exp2(jnp.float32(_EXP2_SCALE) * l)


def _concrete_row_kernel(n_cols, logits_ref, out_ref):
    lane = n_cols // SUB_
    n_full = lane // CHUNK_
    tail = lane - n_full * CHUNK_
    row_base = pl.program_id(0) * n_cols

    s_io = jax.lax.broadcasted_iota(jnp.int32, (SUB_, CHUNK_), 0) * lane
    c_io = jax.lax.broadcasted_iota(jnp.int32, (SUB_, CHUNK_), 1)
    base_idx = s_io + c_io + row_base

    def pass1(i, acc):
        st = i * CHUNK_
        l = logits_ref[0, :, pl.ds(st, CHUNK_)]
        w = _weights(l, (base_idx + st).astype(jnp.uint32))
        out_ref[0, :, pl.ds(st, CHUNK_)] = w
        return acc + w

    acc = jax.lax.fori_loop(
        0, n_full, pass1, jnp.zeros((SUB_, CHUNK_), jnp.float32)
    )
    total = jnp.sum(acc)

    if tail:
        st = n_full * CHUNK_
        l = logits_ref[0, :, pl.ds(st, tail)]
        w = _weights(l, (base_idx[:, :tail] + st).astype(jnp.uint32))
        out_ref[0, :, pl.ds(st, tail)] = w
        total = total + jnp.sum(w)

    inv = jnp.float32(1.0) / total

    def pass2(i, carry):
        st = i * CHUNK_
        out_ref[0, :, pl.ds(st, CHUNK_)] = out_ref[0, :, pl.ds(st, CHUNK_)] * inv
        return carry

    jax.lax.fori_loop(0, n_full, pass2, 0)
    if tail:
        st = n_full * CHUNK_
        out_ref[0, :, pl.ds(st, tail)] = out_ref[0, :, pl.ds(st, tail)] * inv


def kernel(logits):
    rows, n_cols = logits.shape
    lane = n_cols // SUB_
    x3 = logits.reshape(rows, SUB_, lane)
    out = pl.pallas_call(
        functools.partial(_concrete_row_kernel, n_cols),
        grid=(rows,),
        in_specs=[pl.BlockSpec((1, SUB_, lane), lambda r: (r, 0, 0))],
        out_specs=pl.BlockSpec((1, SUB_, lane), lambda r: (r, 0, 0)),
        out_shape=jax.ShapeDtypeStruct((rows, SUB_, lane), jnp.float32),
    )(x3)
    return out.reshape(rows, n_cols)


# speedup vs baseline: 1.3833x; 1.1186x over previous
import functools

import jax
import jax.numpy as jnp
import numpy as np
from jax.experimental import pallas as pl

TAU_ = 0.5
EPS_ = float(np.finfo(np.float32).tiny)
SUB_ = 8
CHUNK_ = 1024
_EXP2_SCALE = float(2.0 / np.log(2.0))


def _weights(l, idx):
    ks0 = jnp.uint32(0)
    ks1 = jnp.uint32(1)
    ks2 = jnp.uint32(0x1BD11BDA ^ 0 ^ 1)

    x0 = jnp.zeros_like(idx)
    x1 = idx + ks1

    def rotl(v, d):
        return (v << jnp.uint32(d)) | (v >> jnp.uint32(32 - d))

    rots_a = (13, 15, 26, 6)
    rots_b = (17, 29, 16, 24)
    inject = ((ks1, ks2), (ks2, ks0), (ks0, ks1), (ks1, ks2), (ks2, ks0))
    for i in range(5):
        for r in (rots_a if i % 2 == 0 else rots_b):
            x0 = x0 + x1
            x1 = rotl(x1, r)
            x1 = x1 ^ x0
        ka, kb = inject[i]
        x0 = x0 + ka
        x1 = x1 + kb + jnp.uint32(i + 1)

    bits = x0 ^ x1

    fu = jax.lax.bitcast_convert_type(
        (bits >> jnp.uint32(9)) | jnp.uint32(0x3F800000), jnp.float32
    ) - jnp.float32(1.0)
    u = jnp.maximum(fu, jnp.float32(EPS_))

    t = jnp.log2(u)
    return (t * t) * jnp.exp2(jnp.float32(_EXP2_SCALE) * l)


UNROLL_ = 3


def _concrete_row_kernel(n_cols, logits_ref, out_ref):
    lane = n_cols // SUB_
    step = UNROLL_ * CHUNK_
    n_outer = lane // step
    row_base = pl.program_id(0) * n_cols

    s_io = jax.lax.broadcasted_iota(jnp.int32, (SUB_, CHUNK_), 0) * lane
    c_io = jax.lax.broadcasted_iota(jnp.int32, (SUB_, CHUNK_), 1)
    base_idx = s_io + c_io + row_base

    def do_chunk(st, width):
        l = logits_ref[0, :, pl.ds(st, width)]
        w = _weights(l, (base_idx[:, :width] + st).astype(jnp.uint32))
        out_ref[0, :, pl.ds(st, width)] = w
        return w

    def pass1(i, accs):
        base = i * step
        return tuple(
            accs[k] + do_chunk(base + k * CHUNK_, CHUNK_)
            for k in range(UNROLL_)
        )

    zeros = jnp.zeros((SUB_, CHUNK_), jnp.float32)
    accs = jax.lax.fori_loop(0, n_outer, pass1, (zeros,) * UNROLL_)
    total = jnp.sum(sum(accs))

    pos = n_outer * step
    while pos + CHUNK_ <= lane:
        total = total + jnp.sum(do_chunk(pos, CHUNK_))
        pos += CHUNK_
    tail = lane - pos
    if tail:
        total = total + jnp.sum(do_chunk(pos, tail))
    n_full = lane // CHUNK_

    inv = jnp.float32(1.0) / total

    def pass2(i, carry):
        st = i * CHUNK_
        out_ref[0, :, pl.ds(st, CHUNK_)] = out_ref[0, :, pl.ds(st, CHUNK_)] * inv
        return carry

    jax.lax.fori_loop(0, n_full, pass2, 0)
    if tail:
        st = n_full * CHUNK_
        out_ref[0, :, pl.ds(st, tail)] = out_ref[0, :, pl.ds(st, tail)] * inv


def kernel(logits):
    rows, n_cols = logits.shape
    lane = n_cols // SUB_
    x3 = logits.reshape(rows, SUB_, lane)
    out = pl.pallas_call(
        functools.partial(_concrete_row_kernel, n_cols),
        grid=(rows,),
        in_specs=[pl.BlockSpec((1, SUB_, lane), lambda r: (r, 0, 0))],
        out_specs=pl.BlockSpec((1, SUB_, lane), lambda r: (r, 0, 0)),
        out_shape=jax.ShapeDtypeStruct((rows, SUB_, lane), jnp.float32),
    )(x3)
    return out.reshape(rows, n_cols)
